# baseline (speedup 1.0000x reference)
import functools

import jax
import jax.numpy as jnp
from jax import lax
from jax.experimental import pallas as pl
from jax.experimental.pallas import tpu as pltpu
from jax.experimental.pallas import tpu_sc as plsc

D_EMB = 16
D_HID = 768
BATCH = 1024
SEQ = 50
NTOK = BATCH * SEQ
NBLK_OUT = NTOK // 8

_info = plsc.get_sparse_core_info()
_NC, _NS = _info.num_cores, _info.num_subcores
_NW = _NC * _NS
_B_PER_W = NTOK // _NW
_CH = 320
_NCH = _B_PER_W // _CH
_G = 16
_NGRP = _CH // _G

_mesh = plsc.VectorSubcoreMesh(core_axis_name="c", subcore_axis_name="s")


@functools.partial(
    pl.kernel,
    out_type=jax.ShapeDtypeStruct((NBLK_OUT, 128), jnp.float32),
    mesh=_mesh,
    scratch_types=[
        pltpu.VMEM((_B_PER_W + _G,), jnp.int32),
        pltpu.VMEM((_G, 8, D_EMB), jnp.float32),
        pltpu.VMEM((_CH // 8, 128), jnp.float32),
        pltpu.SemaphoreType.DMA((_G,)),
    ],
    compiler_params=pltpu.CompilerParams(needs_layout_passes=False),
)
def _sc_gather(table_hbm, idx_hbm, out_hbm, idx_v, ring_v, rows_v, sems):
    wid = lax.axis_index("s") * _NC + lax.axis_index("c")
    base = wid * _B_PER_W
    pltpu.sync_copy(
        idx_hbm.at[pl.ds(base, _B_PER_W)], idx_v.at[pl.ds(0, _B_PER_W)]
    )
    iota = lax.iota(jnp.int32, _G)
    mask7 = jnp.int32(~7)

    for c in range(_NCH):
        cbase = c * _CH
        prow = idx_v[pl.ds(cbase, _G)] & mask7
        for b in range(_G):
            pltpu.async_copy(
                table_hbm.at[pl.ds(pl.multiple_of(prow[b], 8), 8)],
                ring_v.at[b],
                sems.at[b],
            )

        def _body(i, carry, cbase=cbase):
            tloc = i * _G
            lo = idx_v[pl.ds(cbase + tloc, _G)] & jnp.int32(7)
            nrow = idx_v[pl.ds(cbase + tloc + _G, _G)] & mask7
            tvec = tloc + iota
            for b in range(_G):
                pltpu.make_async_copy(
                    table_hbm.at[pl.ds(0, 8)], ring_v.at[b], sems.at[b]
                ).wait()
            rrow = lax.shift_right_logical(tvec, jnp.int32(3))
            rcol = (tvec & jnp.int32(7)) * jnp.int32(D_EMB)
            for col in range(D_EMB):
                cv = jnp.full((_G,), col, jnp.int32)
                vals = plsc.load_gather(ring_v, [iota, lo, cv])
                plsc.store_scatter(rows_v, [rrow, rcol + cv], vals)

            @pl.when(i + 1 < _NGRP)
            def _issue():
                for b in range(_G):
                    pltpu.async_copy(
                        table_hbm.at[pl.ds(pl.multiple_of(nrow[b], 8), 8)],
                        ring_v.at[b],
                        sems.at[b],
                    )

            return carry

        lax.fori_loop(0, _NGRP, _body, 0)
        pltpu.sync_copy(
            rows_v,
            out_hbm.at[pl.ds(pl.multiple_of((base + cbase) // 8, 8), _CH // 8)],
        )


_BLK = 400


def _proj_body(emb_ref, w_ref, b_ref, out_ref):
    out_ref[...] = (
        jnp.dot(emb_ref[...], w_ref[...], preferred_element_type=jnp.float32)
        + b_ref[...]
    )


def _project(emb128, Wbig, bbig):
    return pl.pallas_call(
        _proj_body,
        grid=(NBLK_OUT // _BLK,),
        in_specs=[
            pl.BlockSpec((_BLK, 128), lambda i: (i, 0)),
            pl.BlockSpec((128, 8 * D_HID), lambda i: (0, 0)),
            pl.BlockSpec((1, 8 * D_HID), lambda i: (0, 0)),
        ],
        out_specs=pl.BlockSpec((_BLK, 8 * D_HID), lambda i: (i, 0)),
        out_shape=jax.ShapeDtypeStruct((NBLK_OUT, 8 * D_HID), jnp.float32),
    )(emb128, Wbig, bbig)


def kernel(idxs, table, W, b):
    flat = idxs.reshape(-1)
    emb128 = _sc_gather(table, flat)
    Wbig = jnp.zeros((8, 8, D_EMB, D_HID), W.dtype)
    Wbig = Wbig.at[jnp.arange(8), jnp.arange(8)].set(W[None])
    Wbig = Wbig.transpose(0, 2, 1, 3).reshape(128, 8 * D_HID)
    bbig = jnp.tile(b, 8).reshape(1, 8 * D_HID)
    out = _project(emb128, Wbig, bbig)
    return out.reshape(BATCH, SEQ, D_HID)


# speedup vs baseline: 1.0322x; 1.0322x over previous
import functools

import jax
import jax.numpy as jnp
from jax import lax
from jax.experimental import pallas as pl
from jax.experimental.pallas import tpu as pltpu
from jax.experimental.pallas import tpu_sc as plsc

D_EMB = 16
D_HID = 768
BATCH = 1024
SEQ = 50
NTOK = BATCH * SEQ
NBLK_OUT = NTOK // 8

_info = plsc.get_sparse_core_info()
_NC, _NS = _info.num_cores, _info.num_subcores
_NW = _NC * _NS
_B_PER_W = NTOK // _NW
_CH = 320
_NCH = _B_PER_W // _CH
_G = 16
_NGRP = _CH // _G

_mesh = plsc.VectorSubcoreMesh(core_axis_name="c", subcore_axis_name="s")


@functools.partial(
    pl.kernel,
    out_type=jax.ShapeDtypeStruct((NTOK, 128), jnp.float32),
    mesh=_mesh,
    scratch_types=[
        pltpu.VMEM((_B_PER_W + _G,), jnp.int32),
        pltpu.VMEM((_G, 8, D_EMB), jnp.float32),
        pltpu.VMEM((_CH, 128), jnp.float32),
        pltpu.SemaphoreType.DMA((_G,)),
    ],
    compiler_params=pltpu.CompilerParams(needs_layout_passes=False),
)
def _sc_gather(table_hbm, idx_hbm, out_hbm, idx_v, ring_v, rows_v, sems):
    wid = lax.axis_index("s") * _NC + lax.axis_index("c")
    base = wid * _B_PER_W
    pltpu.sync_copy(
        idx_hbm.at[pl.ds(base, _B_PER_W)], idx_v.at[pl.ds(0, _B_PER_W)]
    )
    iota = lax.iota(jnp.int32, _G)
    mask7 = jnp.int32(~7)
    zeros16 = jnp.zeros((16,), jnp.float32)

    def _zero_body(r, carry):
        for cc in range(1, 8):
            rows_v[r, pl.ds(cc * 16, 16)] = zeros16
        return carry

    lax.fori_loop(0, _CH, _zero_body, 0)

    for c in range(_NCH):
        cbase = c * _CH
        prow = idx_v[pl.ds(cbase, _G)] & mask7
        for b in range(_G):
            pltpu.async_copy(
                table_hbm.at[pl.ds(pl.multiple_of(prow[b], 8), 8)],
                ring_v.at[b],
                sems.at[b],
            )

        def _body(i, carry, cbase=cbase):
            tloc = i * _G
            lo = idx_v[pl.ds(cbase + tloc, _G)] & jnp.int32(7)
            nrow = idx_v[pl.ds(cbase + tloc + _G, _G)] & mask7
            tvec = tloc + iota
            for b in range(_G):
                pltpu.make_async_copy(
                    table_hbm.at[pl.ds(0, 8)], ring_v.at[b], sems.at[b]
                ).wait()
            for col in range(D_EMB):
                cv = jnp.full((_G,), col, jnp.int32)
                vals = plsc.load_gather(ring_v, [iota, lo, cv])
                plsc.store_scatter(rows_v, [tvec, cv], vals)

            @pl.when(i + 1 < _NGRP)
            def _issue():
                for b in range(_G):
                    pltpu.async_copy(
                        table_hbm.at[pl.ds(pl.multiple_of(nrow[b], 8), 8)],
                        ring_v.at[b],
                        sems.at[b],
                    )

            return carry

        lax.fori_loop(0, _NGRP, _body, 0)
        pltpu.sync_copy(
            rows_v,
            out_hbm.at[pl.ds(pl.multiple_of(base + cbase, 8), _CH)],
        )


_BLK = 3200


def _proj_body(emb_ref, w_ref, b_ref, out_ref):
    out_ref[...] = (
        jnp.dot(emb_ref[...], w_ref[...], preferred_element_type=jnp.float32)
        + b_ref[...]
    )


def _project(emb128, Wpad, b2):
    return pl.pallas_call(
        _proj_body,
        grid=(NTOK // _BLK,),
        in_specs=[
            pl.BlockSpec((_BLK, 128), lambda i: (i, 0)),
            pl.BlockSpec((128, D_HID), lambda i: (0, 0)),
            pl.BlockSpec((1, D_HID), lambda i: (0, 0)),
        ],
        out_specs=pl.BlockSpec((_BLK, D_HID), lambda i: (i, 0)),
        out_shape=jax.ShapeDtypeStruct((NTOK, D_HID), jnp.float32),
    )(emb128, Wpad, b2)


def kernel(idxs, table, W, b):
    flat = idxs.reshape(-1)
    emb128 = _sc_gather(table, flat)
    Wpad = jnp.concatenate([W, jnp.zeros((128 - D_EMB, D_HID), W.dtype)])
    out = _project(emb128, Wpad, b.reshape(1, D_HID))
    return out.reshape(BATCH, SEQ, D_HID)


# speedup vs baseline: 1.3071x; 1.2664x over previous
import functools

import jax
import jax.numpy as jnp
from jax import lax
from jax.experimental import pallas as pl
from jax.experimental.pallas import tpu as pltpu
from jax.experimental.pallas import tpu_sc as plsc

D_EMB = 16
D_HID = 768
NTOK = 1024 * 50
NBLK_TBL = 125000
NBLK_OUT = NTOK // 8

_info = plsc.get_sparse_core_info()
_NC, _NS = _info.num_cores, _info.num_subcores
_NW = _NC * _NS
_B_PER_W = NTOK // _NW
_CH = 320
_NCH = _B_PER_W // _CH
_G = 16
_NGRP = _CH // _G

_mesh = plsc.VectorSubcoreMesh(core_axis_name="c", subcore_axis_name="s")


@functools.partial(
    pl.kernel,
    out_type=jax.ShapeDtypeStruct((NBLK_OUT, 8, D_EMB), jnp.float32),
    mesh=_mesh,
    scratch_types=[
        pltpu.VMEM((_B_PER_W + _G,), jnp.int32),
        pltpu.VMEM((_G, 8, D_EMB), jnp.float32),
        pltpu.VMEM((_CH // 8, 8, D_EMB), jnp.float32),
        pltpu.SemaphoreType.DMA((_G,)),
    ],
)
def _sc_gather(table_hbm, idx_hbm, out_hbm, idx_v, ring_v, rows_v, sems):
    wid = lax.axis_index("s") * _NC + lax.axis_index("c")
    base = wid * _B_PER_W
    pltpu.sync_copy(
        idx_hbm.at[pl.ds(base, _B_PER_W)], idx_v.at[pl.ds(0, _B_PER_W)]
    )

    for c in range(_NCH):
        cbase = c * _CH
        pblk = lax.shift_right_logical(idx_v[pl.ds(cbase, _G)], jnp.int32(3))
        for b in range(_G):
            pltpu.async_copy(table_hbm.at[pblk[b]], ring_v.at[b], sems.at[b])

        def _body(i, carry, cbase=cbase):
            tloc = i * _G
            lo = idx_v[pl.ds(cbase + tloc, _G)] & jnp.int32(7)
            nblk = lax.shift_right_logical(
                idx_v[pl.ds(cbase + tloc + _G, _G)], jnp.int32(3)
            )
            for b in range(_G):
                pltpu.make_async_copy(
                    table_hbm.at[0], ring_v.at[b], sems.at[b]
                ).wait()
                rows_v[i * 2 + b // 8, b % 8, pl.ds(0, D_EMB)] = ring_v[
                    b, lo[b], pl.ds(0, D_EMB)
                ]

                @pl.when(i + 1 < _NGRP)
                def _issue(b=b):
                    pltpu.async_copy(
                        table_hbm.at[nblk[b]], ring_v.at[b], sems.at[b]
                    )

            return carry

        lax.fori_loop(0, _NGRP, _body, 0)
        pltpu.sync_copy(
            rows_v, out_hbm.at[pl.ds((base + cbase) // 8, _CH // 8)]
        )


_BLK = 3200


def _proj_body(emb_ref, w_ref, b_ref, out_ref):
    out_ref[...] = (
        jnp.dot(emb_ref[...], w_ref[...], preferred_element_type=jnp.float32)
        + b_ref[...]
    )


def _project(emb, W, b2):
    return pl.pallas_call(
        _proj_body,
        grid=(NTOK // _BLK,),
        in_specs=[
            pl.BlockSpec((_BLK, D_EMB), lambda i: (i, 0)),
            pl.BlockSpec((D_EMB, D_HID), lambda i: (0, 0)),
            pl.BlockSpec((1, D_HID), lambda i: (0, 0)),
        ],
        out_specs=pl.BlockSpec((_BLK, D_HID), lambda i: (i, 0)),
        out_shape=jax.ShapeDtypeStruct((NTOK, D_HID), jnp.float32),
    )(emb, W, b2)


def kernel(idxs, table, W, b):
    B, S = idxs.shape
    flat = idxs.reshape(-1)
    table3 = table.reshape(NBLK_TBL, 8, D_EMB)
    emb3 = _sc_gather(table3, flat)
    emb = emb3.reshape(NTOK, D_EMB)
    out = _project(emb, W, b.reshape(1, D_HID))
    return out.reshape(B, S, D_HID)
